# baseline (speedup 1.0000x reference)
import functools
import jax
import jax.numpy as jnp
from jax import lax
from jax.experimental import pallas as pl
from jax.experimental.pallas import tpu as pltpu
from jax.experimental.pallas import tpu_sc as plsc

_D = 4096
_H = 32
_L = 4
_N = 64
_THR = 0.3
_LN2 = 0.6931471805599453


def _precompute_body(q_ref, ops_ref, wq_ref, bq_ref, wo_ref, bo_ref,
                     aq_ref, gq_ref, sa2_ref, sg2_ref, m_ref):
    qvec = q_ref[...]
    ops = ops_ref[...]
    wq = wq_ref[...]
    wo = wo_ref[...]
    bq_all = bq_ref[...]
    bo_all = bo_ref[...]

    dn = (((1,), (1,)), ((), ()))
    qproj = jax.lax.dot_general(qvec, wq, dn, preferred_element_type=jnp.float32)
    A = jax.lax.dot_general(ops, wo[:, :_D], dn, preferred_element_type=jnp.float32)
    G = jax.lax.dot_general(ops, wo[:, _D:], dn, preferred_element_type=jnp.float32)

    ones_h = jnp.ones((1, _H), dtype=jnp.float32)
    aq_rows, gq_rows, sa2_rows, sg2_rows, m_blocks = [], [], [], [], []
    for l in range(_L):
        qs = qproj[:, l * _H:(l + 1) * _H] + bq_all[l:l + 1, :]
        qn = qs / jnp.maximum(jnp.sqrt(jnp.sum(qs * qs)), 1e-12)
        Ap = A[:, l * _H:(l + 1) * _H] + bo_all[l:l + 1, :]
        Gl = G[:, l * _H:(l + 1) * _H]
        aq_rows.append(jax.lax.dot_general(qn, Ap, dn, preferred_element_type=jnp.float32))
        gq_rows.append(jax.lax.dot_general(qn, Gl, dn, preferred_element_type=jnp.float32))
        sa2_rows.append(jax.lax.dot_general(ones_h, Ap * Ap, dn, preferred_element_type=jnp.float32))
        sg2_rows.append(jax.lax.dot_general(ones_h, Gl * Gl, dn, preferred_element_type=jnp.float32))
        if l > 0:
            m_blocks.append(jax.lax.dot_general(Gl, Ap, dn, preferred_element_type=jnp.float32))

    aq_ref[...] = jnp.concatenate(aq_rows, axis=0)
    gq_ref[...] = jnp.concatenate(gq_rows, axis=0)
    sa2_ref[...] = jnp.concatenate(sa2_rows, axis=0)
    sg2_ref[...] = jnp.concatenate(sg2_rows, axis=0)
    m_ref[...] = jnp.concatenate(m_blocks, axis=0)


def _rsqrt_nr(x):
    bits = lax.bitcast_convert_type(x, jnp.int32)
    y = lax.bitcast_convert_type(jnp.int32(0x5F3759DF) - (bits >> 1), jnp.float32)
    for _ in range(3):
        y = y * (1.5 - 0.5 * x * y * y)
    return y


def _log_nr(x):
    bits = lax.bitcast_convert_type(x, jnp.int32)
    ex = ((bits >> 23) & jnp.int32(0xFF)) - 127
    mant = lax.bitcast_convert_type((bits & jnp.int32(0x007FFFFF)) | jnp.int32(0x3F800000),
                                    jnp.float32)
    y = ex.astype(jnp.float32) * _LN2 + (mant - 1.0)
    for _ in range(4):
        y = y + x * jnp.exp(-y) - 1.0
    return y


def _vgather(x, idx):
    dnums = lax.GatherDimensionNumbers(offset_dims=(), collapsed_slice_dims=(0,),
                                       start_index_map=(0,))
    return lax.gather(x, idx[:, None], dnums, slice_sizes=(1,),
                      mode=lax.GatherScatterMode.PROMISE_IN_BOUNDS)


def _sc_select_body(aq_hbm, gq_hbm, sa2_hbm, sg2_hbm, m_hbm, logp_hbm, probs_hbm,
                    aq_v, gq_v, sa2_v, sg2_v, m_v, logp_v, probs_v):
    cid = lax.axis_index("c")
    sid = lax.axis_index("s")

    @pl.when(jnp.logical_and(cid == 0, sid == 0))
    def _():
        pltpu.sync_copy(aq_hbm, aq_v.at[pl.ds(0, _L * _N)])
        pltpu.sync_copy(gq_hbm, gq_v.at[pl.ds(0, _L * _N)])
        pltpu.sync_copy(sa2_hbm, sa2_v.at[pl.ds(0, _L * _N)])
        pltpu.sync_copy(sg2_hbm, sg2_v.at[pl.ds(0, _L * _N)])
        pltpu.sync_copy(m_hbm, m_v)

        iota = lax.iota(jnp.int32, 16)

        def red(v, op):
            for sh in (8, 4, 2, 1):
                v = op(v, _vgather(v, iota ^ sh))
            return v

        def add(a, b):
            return a + b

        big = jnp.broadcast_to(jnp.int32(_N), (16,))
        onef = jnp.broadcast_to(jnp.float32(1.0), (16,))
        zerof = jnp.broadcast_to(jnp.float32(0.0), (16,))
        fs = jnp.int32(0)
        for l in range(_L):
            sr = [aq_v[pl.ds(l * _N + 16 * j, 16)] for j in range(4)]
            ss = [sa2_v[pl.ds(l * _N + 16 * j, 16)] for j in range(4)]
            if l > 0:
                gq_f = jnp.broadcast_to(gq_v[pl.ds(l * _N + fs, 16)][0], (16,))
                sg_f = jnp.broadcast_to(sg2_v[pl.ds(l * _N + fs, 16)][0], (16,))
                mbase = (l - 1) * _N * _N + fs * _N
                for j in range(4):
                    mcol = m_v[pl.ds(mbase + 16 * j, 16)]
                    ss[j] = ss[j] + 2.0 * mcol + sg_f
                    sr[j] = sr[j] + gq_f
            sc = [sr[j] * _rsqrt_nr(jnp.maximum(ss[j], 1e-24)) for j in range(4)]
            mx = red(jnp.maximum(jnp.maximum(sc[0], sc[1]),
                                 jnp.maximum(sc[2], sc[3])), jnp.maximum)
            e = [jnp.exp(sc[j] - mx) for j in range(4)]
            sv = red(e[0] + e[1] + e[2] + e[3], add)
            p = [e[j] / sv for j in range(4)]
            logs = _log_nr(sv)
            lp = [sc[j] - mx - logs for j in range(4)]
            maskf = [jnp.where(p[j] > _THR, onef, zerof) for j in range(4)]
            cnt = red(maskf[0] + maskf[1] + maskf[2] + maskf[3], add)
            has_any = cnt > 0.0
            pmax = red(jnp.maximum(jnp.maximum(p[0], p[1]),
                                   jnp.maximum(p[2], p[3])), jnp.maximum)
            am = red(jnp.minimum(
                jnp.minimum(jnp.where(p[0] == pmax, iota, big),
                            jnp.where(p[1] == pmax, iota + 16, big)),
                jnp.minimum(jnp.where(p[2] == pmax, iota + 32, big),
                            jnp.where(p[3] == pmax, iota + 48, big))),
                jnp.minimum)
            fm = red(jnp.minimum(
                jnp.minimum(jnp.where(maskf[0] > 0.0, iota, big),
                            jnp.where(maskf[1] > 0.0, iota + 16, big)),
                jnp.minimum(jnp.where(maskf[2] > 0.0, iota + 32, big),
                            jnp.where(maskf[3] > 0.0, iota + 48, big))),
                jnp.minimum)
            sel = [jnp.where(has_any, maskf[j],
                             jnp.where((iota + 16 * j) == am, onef, zerof))
                   for j in range(4)]
            llp = red(sel[0] * lp[0] + sel[1] * lp[1] + sel[2] * lp[2] + sel[3] * lp[3],
                      add)
            fs = jnp.where(has_any, fm, am)[0]
            for j in range(4):
                probs_v[pl.ds(l * _N + 16 * j, 16)] = p[j]
            logp_v[pl.ds(l * 16, 16)] = llp

        pltpu.sync_copy(logp_v, logp_hbm)
        pltpu.sync_copy(probs_v, probs_hbm)


_sc_select_cache = []


def _get_sc_select():
    if not _sc_select_cache:
        _sc_select_cache.append(functools.partial(
            pl.kernel,
            out_type=(
                jax.ShapeDtypeStruct((_L * 16,), jnp.float32),
                jax.ShapeDtypeStruct((_L * _N,), jnp.float32),
            ),
            mesh=plsc.VectorSubcoreMesh(core_axis_name="c", subcore_axis_name="s"),
            scratch_types=[
                pltpu.VMEM((_L * _N,), jnp.float32),
                pltpu.VMEM((_L * _N + 16,), jnp.float32),
                pltpu.VMEM((_L * _N,), jnp.float32),
                pltpu.VMEM((_L * _N + 16,), jnp.float32),
                pltpu.VMEM(((_L - 1) * _N * _N,), jnp.float32),
                pltpu.VMEM((_L * 16,), jnp.float32),
                pltpu.VMEM((_L * _N,), jnp.float32),
            ],
        )(_sc_select_body))
    return _sc_select_cache[0]


def kernel(query_embed, operators_embedding, Wq, bq, Wo, bo):
    wq_flat = Wq.reshape(_L * _H, _D)
    wo_flat = Wo.reshape(_L * _H, 2 * _D)
    aq, gq, sa2, sg2, m = pl.pallas_call(
        _precompute_body,
        out_shape=(
            jax.ShapeDtypeStruct((_L, _N), jnp.float32),
            jax.ShapeDtypeStruct((_L, _N), jnp.float32),
            jax.ShapeDtypeStruct((_L, _N), jnp.float32),
            jax.ShapeDtypeStruct((_L, _N), jnp.float32),
            jax.ShapeDtypeStruct(((_L - 1) * _N, _N), jnp.float32),
        ),
    )(query_embed, operators_embedding, wq_flat, bq, wo_flat, bo)
    logp16, probs_flat = _get_sc_select()(aq.reshape(-1), gq.reshape(-1), sa2.reshape(-1),
                                          sg2.reshape(-1), m.reshape(-1))
    return (logp16.reshape(_L, 16)[:, 0], probs_flat.reshape(_L, _N))


# speedup vs baseline: 2.7592x; 2.7592x over previous
import jax
import jax.numpy as jnp
from jax.experimental import pallas as pl
from jax.experimental.pallas import tpu as pltpu

_D = 4096
_H = 32
_L = 4
_N = 64
_THR = 0.3
_CHUNK = 512
_STEPS = _D // _CHUNK


def _body(q_ref, ops_ref, wq_ref, woa_ref, wog_ref, bq_ref, bo_ref,
          logp_ref, probs_ref, qa, Aacc, Gacc):
    c = pl.program_id(0)
    dn = (((1,), (1,)), ((), ()))

    @pl.when(c == 0)
    def _():
        qa[...] = jnp.zeros_like(qa)
        Aacc[...] = jnp.zeros_like(Aacc)
        Gacc[...] = jnp.zeros_like(Gacc)

    qa[...] += jax.lax.dot_general(q_ref[...], wq_ref[...], dn,
                                   preferred_element_type=jnp.float32)
    Aacc[...] += jax.lax.dot_general(ops_ref[...], woa_ref[...], dn,
                                     preferred_element_type=jnp.float32)
    Gacc[...] += jax.lax.dot_general(ops_ref[...], wog_ref[...], dn,
                                     preferred_element_type=jnp.float32)

    @pl.when(c == _STEPS - 1)
    def _():
        qproj = qa[...]
        A = Aacc[...]
        G = Gacc[...]
        bq_all = bq_ref[...]
        bo_all = bo_ref[...]

        row_iota = jax.lax.broadcasted_iota(jnp.int32, (_N, 1), 0)
        col_iota = jax.lax.broadcasted_iota(jnp.int32, (1, _N), 1)

        first_idx = jnp.int32(0)
        logp_rows = []
        probs_rows = []
        for l in range(_L):
            qs = qproj[:, l * _H:(l + 1) * _H] + bq_all[l:l + 1, :]
            qn = qs / jnp.maximum(jnp.sqrt(jnp.sum(qs * qs)), 1e-12)
            opsl = A[:, l * _H:(l + 1) * _H] + bo_all[l:l + 1, :]
            if l > 0:
                gmask = (row_iota == first_idx).astype(jnp.float32)
                grow = jnp.sum(G[:, l * _H:(l + 1) * _H] * gmask, axis=0, keepdims=True)
                opsl = opsl + grow
            rn = jnp.maximum(jnp.sqrt(jnp.sum(opsl * opsl, axis=1, keepdims=True)), 1e-12)
            opsn = opsl / rn
            scores = jax.lax.dot_general(qn, opsn, dn, preferred_element_type=jnp.float32)
            m = jnp.max(scores)
            e = jnp.exp(scores - m)
            s = jnp.sum(e)
            probs = e / s
            logp = scores - m - jnp.log(s)
            mask = probs > _THR
            has_any = jnp.sum(mask.astype(jnp.float32)) > 0.0
            pmax = jnp.max(probs)
            am = jnp.min(jnp.where(probs == pmax, col_iota, _N))
            sel = jnp.where(has_any, mask.astype(jnp.float32),
                            (col_iota == am).astype(jnp.float32))
            llp = jnp.sum(logp * sel)
            fm = jnp.min(jnp.where(mask, col_iota, _N))
            first_idx = jnp.where(has_any, fm, am)
            logp_rows.append(jnp.broadcast_to(llp[None, None], (1, 1)))
            probs_rows.append(probs)

        logp_ref[...] = jnp.concatenate(logp_rows, axis=0)
        probs_ref[...] = jnp.concatenate(probs_rows, axis=0)


def kernel(query_embed, operators_embedding, Wq, bq, Wo, bo):
    wq_flat = Wq.reshape(_L * _H, _D)
    wo_flat = Wo.reshape(_L * _H, 2 * _D)
    logp, probs = pl.pallas_call(
        _body,
        grid=(_STEPS,),
        in_specs=[
            pl.BlockSpec((1, _CHUNK), lambda c: (0, c)),
            pl.BlockSpec((_N, _CHUNK), lambda c: (0, c)),
            pl.BlockSpec((_L * _H, _CHUNK), lambda c: (0, c)),
            pl.BlockSpec((_L * _H, _CHUNK), lambda c: (0, c)),
            pl.BlockSpec((_L * _H, _CHUNK), lambda c: (0, c + _STEPS)),
            pl.BlockSpec((_L, _H), lambda c: (0, 0)),
            pl.BlockSpec((_L, _H), lambda c: (0, 0)),
        ],
        out_specs=(
            pl.BlockSpec((_L, 1), lambda c: (0, 0)),
            pl.BlockSpec((_L, _N), lambda c: (0, 0)),
        ),
        out_shape=(
            jax.ShapeDtypeStruct((_L, 1), jnp.float32),
            jax.ShapeDtypeStruct((_L, _N), jnp.float32),
        ),
        scratch_shapes=[
            pltpu.VMEM((1, _L * _H), jnp.float32),
            pltpu.VMEM((_N, _L * _H), jnp.float32),
            pltpu.VMEM((_N, _L * _H), jnp.float32),
        ],
    )(query_embed, operators_embedding, wq_flat, wo_flat, wo_flat, bq, bo)
    return (logp[:, 0], probs)


# speedup vs baseline: 3.2777x; 1.1879x over previous
import jax
import jax.numpy as jnp
from jax.experimental import pallas as pl
from jax.experimental.pallas import tpu as pltpu

_D = 4096
_H = 32
_L = 4
_N = 64
_THR = 0.3
_CHUNK = 1024
_STEPS = _D // _CHUNK


def _body(q_ref, ops_ref, wq_ref, woa_ref, wog_ref, bq_ref, bo_ref,
          logp_ref, probs_ref, qa, Aacc, Gacc):
    c = pl.program_id(0)
    dn = (((1,), (1,)), ((), ()))

    @pl.when(c == 0)
    def _():
        qa[...] = jnp.zeros_like(qa)
        Aacc[...] = jnp.zeros_like(Aacc)
        Gacc[...] = jnp.zeros_like(Gacc)

    qa[...] += jax.lax.dot_general(q_ref[...], wq_ref[...], dn,
                                   preferred_element_type=jnp.float32)
    Aacc[...] += jax.lax.dot_general(ops_ref[...], woa_ref[...], dn,
                                     preferred_element_type=jnp.float32)
    Gacc[...] += jax.lax.dot_general(ops_ref[...], wog_ref[...], dn,
                                     preferred_element_type=jnp.float32)

    @pl.when(c == _STEPS - 1)
    def _():
        qproj = qa[...]
        A = Aacc[...]
        G = Gacc[...]
        bq_all = bq_ref[...]
        bo_all = bo_ref[...]

        row_iota = jax.lax.broadcasted_iota(jnp.int32, (_N, 1), 0)
        col_iota = jax.lax.broadcasted_iota(jnp.int32, (1, _N), 1)

        first_idx = jnp.int32(0)
        logp_rows = []
        probs_rows = []
        for l in range(_L):
            qs = qproj[:, l * _H:(l + 1) * _H] + bq_all[l:l + 1, :]
            qn = qs / jnp.maximum(jnp.sqrt(jnp.sum(qs * qs)), 1e-12)
            opsl = A[:, l * _H:(l + 1) * _H] + bo_all[l:l + 1, :]
            if l > 0:
                gmask = (row_iota == first_idx).astype(jnp.float32)
                grow = jnp.sum(G[:, l * _H:(l + 1) * _H] * gmask, axis=0, keepdims=True)
                opsl = opsl + grow
            rn = jnp.maximum(jnp.sqrt(jnp.sum(opsl * opsl, axis=1, keepdims=True)), 1e-12)
            opsn = opsl / rn
            scores = jax.lax.dot_general(qn, opsn, dn, preferred_element_type=jnp.float32)
            m = jnp.max(scores)
            e = jnp.exp(scores - m)
            s = jnp.sum(e)
            probs = e / s
            logp = scores - m - jnp.log(s)
            mask = probs > _THR
            has_any = jnp.sum(mask.astype(jnp.float32)) > 0.0
            pmax = jnp.max(probs)
            am = jnp.min(jnp.where(probs == pmax, col_iota, _N))
            sel = jnp.where(has_any, mask.astype(jnp.float32),
                            (col_iota == am).astype(jnp.float32))
            llp = jnp.sum(logp * sel)
            fm = jnp.min(jnp.where(mask, col_iota, _N))
            first_idx = jnp.where(has_any, fm, am)
            logp_rows.append(jnp.broadcast_to(llp[None, None], (1, 1)))
            probs_rows.append(probs)

        logp_ref[...] = jnp.concatenate(logp_rows, axis=0)
        probs_ref[...] = jnp.concatenate(probs_rows, axis=0)


def kernel(query_embed, operators_embedding, Wq, bq, Wo, bo):
    wq_flat = Wq.reshape(_L * _H, _D)
    wo_flat = Wo.reshape(_L * _H, 2 * _D)
    logp, probs = pl.pallas_call(
        _body,
        grid=(_STEPS,),
        in_specs=[
            pl.BlockSpec((1, _CHUNK), lambda c: (0, c)),
            pl.BlockSpec((_N, _CHUNK), lambda c: (0, c)),
            pl.BlockSpec((_L * _H, _CHUNK), lambda c: (0, c)),
            pl.BlockSpec((_L * _H, _CHUNK), lambda c: (0, c)),
            pl.BlockSpec((_L * _H, _CHUNK), lambda c: (0, c + _STEPS)),
            pl.BlockSpec((_L, _H), lambda c: (0, 0)),
            pl.BlockSpec((_L, _H), lambda c: (0, 0)),
        ],
        out_specs=(
            pl.BlockSpec((_L, 1), lambda c: (0, 0)),
            pl.BlockSpec((_L, _N), lambda c: (0, 0)),
        ),
        out_shape=(
            jax.ShapeDtypeStruct((_L, 1), jnp.float32),
            jax.ShapeDtypeStruct((_L, _N), jnp.float32),
        ),
        scratch_shapes=[
            pltpu.VMEM((1, _L * _H), jnp.float32),
            pltpu.VMEM((_N, _L * _H), jnp.float32),
            pltpu.VMEM((_N, _L * _H), jnp.float32),
        ],
    )(query_embed, operators_embedding, wq_flat, wo_flat, wo_flat, bq, bo)
    return (logp[:, 0], probs)
